# baseline (speedup 1.0000x reference)
import jax
import jax.numpy as jnp
from jax.experimental import pallas as pl
from jax.experimental.pallas import tpu as pltpu


def _fused_kernel(x_ref, o_ref):
    m = x_ref.shape[1]
    x = x_ref[0]
    norm = jnp.sqrt(jnp.sum(x * x, axis=-1, keepdims=True))
    xn = x / jnp.maximum(norm, 1e-12)
    h = jax.lax.dot_general(xn * 0.5, xn, (((1,), (1,)), ((), ())),
                            preferred_element_type=jnp.float32)
    row = jax.lax.broadcasted_iota(jnp.int32, h.shape, 0)
    col = jax.lax.broadcasted_iota(jnp.int32, h.shape, 1)
    h = jnp.where(row == col, 0.0, h)
    a = h[0:128, :]
    b = jnp.full_like(a, -3.0)
    for g in range(1, m // 128):
        v = h[g * 128:(g + 1) * 128, :]
        b = jnp.maximum(b, jnp.minimum(a, v))
        a = jnp.maximum(a, v)
    m1 = jnp.max(a, axis=0, keepdims=True)
    m2a = jnp.max(jnp.where(a == m1, -3.0, a), axis=0, keepdims=True)
    t_col = jnp.maximum(m2a, jnp.max(b, axis=0, keepdims=True))
    t_row = jnp.swapaxes(t_col, 0, 1)
    o_ref[0] = (jnp.where(h >= t_row, h, 0.0)
                + jnp.where(h >= t_col, h, 0.0))


def kernel(x, W1, b1, W2, b2):
    b, m, d = x.shape
    return pl.pallas_call(
        _fused_kernel,
        grid=(b,),
        in_specs=[pl.BlockSpec((1, m, d), lambda i: (i, 0, 0))],
        out_specs=pl.BlockSpec((1, m, m), lambda i: (i, 0, 0)),
        out_shape=jax.ShapeDtypeStruct((b, m, m), jnp.float32),
        compiler_params=pltpu.CompilerParams(
            dimension_semantics=("arbitrary",)),
    )(x)


# speedup vs baseline: 1.0318x; 1.0318x over previous
import functools

import jax
import jax.numpy as jnp
from jax.experimental import pallas as pl
from jax.experimental.pallas import tpu as pltpu


def _fused_kernel(rows, x_ref, o_ref, h_s, tc_s, tr_s):
    j = pl.program_id(1)
    m = x_ref.shape[1]

    @pl.when(j == 0)
    def _prologue():
        x = x_ref[0]
        norm = jnp.sqrt(jnp.sum(x * x, axis=-1, keepdims=True))
        xn = x / jnp.maximum(norm, 1e-12)
        h = jax.lax.dot_general(xn * 0.5, xn, (((1,), (1,)), ((), ())),
                                preferred_element_type=jnp.float32)
        row = jax.lax.broadcasted_iota(jnp.int32, h.shape, 0)
        col = jax.lax.broadcasted_iota(jnp.int32, h.shape, 1)
        h = jnp.where(row == col, 0.0, h)
        h_s[...] = h
        a = h[0:128, :]
        b = jnp.full_like(a, -3.0)
        for g in range(1, m // 128):
            v = h[g * 128:(g + 1) * 128, :]
            b = jnp.maximum(b, jnp.minimum(a, v))
            a = jnp.maximum(a, v)
        m1 = jnp.max(a, axis=0, keepdims=True)
        m2a = jnp.max(jnp.where(a == m1, -3.0, a), axis=0, keepdims=True)
        t = jnp.maximum(m2a, jnp.max(b, axis=0, keepdims=True))
        tc_s[...] = t
        tr_s[...] = jnp.swapaxes(t, 0, 1)

    h_j = h_s[pl.ds(j * rows, rows), :]
    t_row = tr_s[pl.ds(j * rows, rows), :]
    t_col = tc_s[...]
    o_ref[0] = (jnp.where(h_j >= t_row, h_j, 0.0)
                + jnp.where(h_j >= t_col, h_j, 0.0))


def kernel(x, W1, b1, W2, b2):
    b, m, d = x.shape
    rows = 2048

    return pl.pallas_call(
        functools.partial(_fused_kernel, rows),
        grid=(b, m // rows),
        in_specs=[pl.BlockSpec((1, m, d), lambda i, j: (i, 0, 0))],
        out_specs=pl.BlockSpec((1, rows, m), lambda i, j: (i, j, 0)),
        out_shape=jax.ShapeDtypeStruct((b, m, m), jnp.float32),
        scratch_shapes=[
            pltpu.VMEM((m, m), jnp.float32),
            pltpu.VMEM((1, m), jnp.float32),
            pltpu.VMEM((m, 1), jnp.float32),
        ],
        compiler_params=pltpu.CompilerParams(
            dimension_semantics=("arbitrary", "arbitrary")),
    )(x)
